# baseline (speedup 1.0000x reference)
import jax
import jax.numpy as jnp
from jax.experimental import pallas as pl
from jax.experimental.pallas import tpu as pltpu

N = 1024
RU = 64
B = 32
M = 3
BB = 4
GRID = B // BB
N2 = N // 2
F32 = jnp.float32
BF16 = jnp.bfloat16
DN = (((0,), (0,)), ((), ()))


def _h_to_nodemajor(hb):
    t = jnp.transpose(hb.reshape(BB, N2, 2 * RU), (1, 0, 2))
    p0 = t[:, :, :RU].reshape(N2, BB * RU)
    p1 = t[:, :, RU:].reshape(N2, BB * RU)
    return jnp.stack([p0, p1], axis=1).reshape(N, BB * RU)


def _h_to_batchmajor(nh):
    s = nh.reshape(N2, 2, BB * RU)
    q0 = s[:, 0, :].reshape(N2, BB, RU)
    q1 = s[:, 1, :].reshape(N2, BB, RU)
    q = jnp.concatenate([q0, q1], axis=2)
    return jnp.transpose(q, (1, 0, 2)).reshape(BB, N * RU)


def _body(adj_ref, xin_ref, h_ref, whg_ref, wig_ref, bg_ref,
          whc_ref, wic_ref, bc_ref, wp_ref, bp_ref,
          out_ref, hout_ref,
          sup_s, x_s, wg_s, wc_s, wp_s, bg_s, bc_s):
    i = pl.program_id(0)

    @pl.when(i == 0)
    def _prep():
        a = adj_ref[...]
        d = jnp.sum(a, axis=1, keepdims=True)
        sup_s[...] = (a * (1.0 / d)).astype(BF16)
        S = sup_s[...]
        x0f = xin_ref[...].T
        x0 = x0f.astype(BF16)
        x1 = jax.lax.dot_general(S, x0, DN, preferred_element_type=F32)
        x1b = x1.astype(BF16)
        x2b = (2.0 * jax.lax.dot_general(S, x1b, DN, preferred_element_type=F32)
               - x0f).astype(BF16)
        for j in range(GRID):
            sl = slice(j * BB, (j + 1) * BB)
            x_s[j, :, 0 * BB:1 * BB] = x0[:, sl]
            x_s[j, :, 1 * BB:2 * BB] = x1b[:, sl]
            x_s[j, :, 2 * BB:3 * BB] = x2b[:, sl]

        wg_s[...] = jnp.zeros((M * BB * (RU + 1), 2 * BB * RU), BF16)
        wc_s[...] = jnp.zeros((M * BB * (RU + 1), BB * RU), BF16)
        wp_s[...] = jnp.zeros((BB * RU, BB), BF16)
        for m in range(M):
            hsl = slice(m * RU, (m + 1) * RU)
            xrow = M * BB * RU + m * BB
            for b in range(BB):
                rows = slice(m * BB * RU + b * RU, m * BB * RU + (b + 1) * RU)
                csl = slice(b * RU, (b + 1) * RU)
                usl = slice(BB * RU + b * RU, BB * RU + (b + 1) * RU)
                wg_s[rows, csl] = whg_ref[hsl, :RU]
                wg_s[rows, usl] = whg_ref[hsl, RU:]
                wc_s[rows, csl] = whc_ref[hsl, :]
                wg_s[xrow + b:xrow + b + 1, csl] = wig_ref[m:m + 1, :RU]
                wg_s[xrow + b:xrow + b + 1, usl] = wig_ref[m:m + 1, RU:]
                wc_s[xrow + b:xrow + b + 1, csl] = wic_ref[m:m + 1, :]
        for b in range(BB):
            csl = slice(b * RU, (b + 1) * RU)
            usl = slice(BB * RU + b * RU, BB * RU + (b + 1) * RU)
            wp_s[csl, b:b + 1] = wp_ref[...]
            bg_s[0:1, csl] = bg_ref[:, :RU]
            bg_s[0:1, usl] = bg_ref[:, RU:]
            bc_s[0:1, csl] = bc_ref[...]

    S = sup_s[...]

    def spmm(x):
        return jax.lax.dot_general(S, x, DN, preferred_element_type=F32)

    H0 = _h_to_nodemajor(h_ref[0])
    H0b = H0.astype(BF16)
    H1 = spmm(H0b)
    H1b = H1.astype(BF16)
    H2 = 2.0 * spmm(H1b) - H0
    xt = x_s[i]
    Xg = jnp.concatenate([H0b, H1b, H2.astype(BF16), xt], axis=1)
    value = jax.nn.sigmoid(
        jnp.dot(Xg, wg_s[...], preferred_element_type=F32) + bg_s[...])
    r = value[:, :BB * RU]
    u = value[:, BB * RU:]

    rH = r * H0
    rHb = rH.astype(BF16)
    R1 = spmm(rHb)
    R1b = R1.astype(BF16)
    R2 = 2.0 * spmm(R1b) - rH
    Xc = jnp.concatenate([rHb, R1b, R2.astype(BF16), xt], axis=1)
    c = jnp.tanh(
        jnp.dot(Xc, wc_s[...], preferred_element_type=F32) + bc_s[...])

    nh = u * H0 + (1.0 - u) * c
    hout_ref[0] = _h_to_batchmajor(nh)
    pj = jnp.dot(nh.astype(BF16), wp_s[...], preferred_element_type=F32)
    out_ref[0] = pj.T + bp_ref[...]


def kernel(inputs, hidden_state, adj, W_gate, b_gate, W_cand, b_cand,
           W_proj, b_proj):
    h2 = hidden_state[0].reshape(GRID, BB, N * RU)

    wg3 = W_gate.reshape(RU + 1, M, 2 * RU)
    whg = wg3[1:].transpose(1, 0, 2).reshape(M * RU, 2 * RU).astype(BF16)
    wig = wg3[0].astype(BF16)
    wc3 = W_cand.reshape(RU + 1, M, RU)
    whc = wc3[1:].transpose(1, 0, 2).reshape(M * RU, RU).astype(BF16)
    wic = wc3[0].astype(BF16)

    const = lambda i: (0, 0)
    out_bn, hout2 = pl.pallas_call(
        _body,
        grid=(GRID,),
        in_specs=[
            pl.BlockSpec((N, N), const),
            pl.BlockSpec((B, N), const),
            pl.BlockSpec((1, BB, N * RU), lambda i: (i, 0, 0)),
            pl.BlockSpec((M * RU, 2 * RU), const),
            pl.BlockSpec((M, 2 * RU), const),
            pl.BlockSpec((1, 2 * RU), const),
            pl.BlockSpec((M * RU, RU), const),
            pl.BlockSpec((M, RU), const),
            pl.BlockSpec((1, RU), const),
            pl.BlockSpec((RU, 1), const),
            pl.BlockSpec((1, 1), const),
        ],
        out_specs=[
            pl.BlockSpec((1, BB, N), lambda i: (i, 0, 0)),
            pl.BlockSpec((1, BB, N * RU), lambda i: (i, 0, 0)),
        ],
        out_shape=[
            jax.ShapeDtypeStruct((GRID, BB, N), F32),
            jax.ShapeDtypeStruct((GRID, BB, N * RU), F32),
        ],
        scratch_shapes=[
            pltpu.VMEM((N, N), BF16),
            pltpu.VMEM((GRID, N, M * BB), BF16),
            pltpu.VMEM((M * BB * (RU + 1), 2 * BB * RU), BF16),
            pltpu.VMEM((M * BB * (RU + 1), BB * RU), BF16),
            pltpu.VMEM((BB * RU, BB), BF16),
            pltpu.VMEM((1, 2 * BB * RU), F32),
            pltpu.VMEM((1, BB * RU), F32),
        ],
    )(adj, inputs, h2, whg, wig, b_gate.reshape(1, 2 * RU),
      whc, wic, b_cand.reshape(1, RU), W_proj.astype(BF16),
      b_proj.reshape(1, 1))

    return out_bn.reshape(B, N), hout2.reshape(1, B, N * RU)


# speedup vs baseline: 1.2579x; 1.2579x over previous
import jax
import jax.numpy as jnp
from jax.experimental import pallas as pl
from jax.experimental.pallas import tpu as pltpu

N = 1024
RU = 64
B = 32
M = 3
BB = 4
GRID = B // BB
N2 = N // 2
F32 = jnp.float32
BF16 = jnp.bfloat16
DN = (((0,), (0,)), ((), ()))


def _to3d_body(h2_ref, o_ref):
    f = h2_ref[...].reshape(8 * N2, 2 * RU)
    g = f.reshape(8 * N2, 2, RU)
    h = g.reshape(8 * N, RU)
    o_ref[...] = h.reshape(8, N, RU)


def _to2d_body(h3_ref, o_ref):
    f = h3_ref[...].reshape(8 * N, RU)
    g = f.reshape(8 * N2, 2, RU)
    h = g.reshape(8 * N2, 2 * RU)
    o_ref[...] = h.reshape(8, N * RU)


def _body(adj_ref, xin_ref, h_ref, whg_ref, wig_ref, bg_ref,
          whc_ref, wic_ref, bc_ref, wp_ref, bp_ref,
          out_ref, hout_ref,
          sup_s, x_s, wg_s, wc_s, wp_s, bg_s, bc_s):
    i = pl.program_id(0)

    @pl.when(i == 0)
    def _prep():
        a = adj_ref[...]
        d = jnp.sum(a, axis=1, keepdims=True)
        sup_s[...] = (a * (1.0 / d)).astype(BF16)
        S = sup_s[...]
        x0f = xin_ref[...]
        x0 = x0f.astype(BF16)
        x1 = jax.lax.dot_general(S, x0, DN, preferred_element_type=F32)
        x1b = x1.astype(BF16)
        x2b = (2.0 * jax.lax.dot_general(S, x1b, DN, preferred_element_type=F32)
               - x0f).astype(BF16)
        for j in range(GRID):
            sl = slice(j * BB, (j + 1) * BB)
            x_s[j, :, 0 * BB:1 * BB] = x0[:, sl]
            x_s[j, :, 1 * BB:2 * BB] = x1b[:, sl]
            x_s[j, :, 2 * BB:3 * BB] = x2b[:, sl]

        wg_s[...] = jnp.zeros((M * BB * (RU + 1), 2 * BB * RU), BF16)
        wc_s[...] = jnp.zeros((M * BB * (RU + 1), BB * RU), BF16)
        wp_s[...] = jnp.zeros((BB * RU, BB), BF16)
        for m in range(M):
            hsl = slice(m * RU, (m + 1) * RU)
            xrow = M * BB * RU + m * BB
            for b in range(BB):
                rows = slice(m * BB * RU + b * RU, m * BB * RU + (b + 1) * RU)
                csl = slice(b * RU, (b + 1) * RU)
                usl = slice(BB * RU + b * RU, BB * RU + (b + 1) * RU)
                wg_s[rows, csl] = whg_ref[hsl, :RU]
                wg_s[rows, usl] = whg_ref[hsl, RU:]
                wc_s[rows, csl] = whc_ref[hsl, :]
                wg_s[xrow + b:xrow + b + 1, csl] = wig_ref[m:m + 1, :RU]
                wg_s[xrow + b:xrow + b + 1, usl] = wig_ref[m:m + 1, RU:]
                wc_s[xrow + b:xrow + b + 1, csl] = wic_ref[m:m + 1, :]
        for b in range(BB):
            csl = slice(b * RU, (b + 1) * RU)
            usl = slice(BB * RU + b * RU, BB * RU + (b + 1) * RU)
            wp_s[csl, b:b + 1] = wp_ref[...]
            bg_s[0:1, csl] = bg_ref[:, :RU]
            bg_s[0:1, usl] = bg_ref[:, RU:]
            bc_s[0:1, csl] = bc_ref[...]

    S = sup_s[...]

    def spmm(x):
        return jax.lax.dot_general(S, x, DN, preferred_element_type=F32)

    H0 = jnp.concatenate([h_ref[b] for b in range(BB)], axis=1)
    H0b = H0.astype(BF16)
    H1 = spmm(H0b)
    H1b = H1.astype(BF16)
    H2 = 2.0 * spmm(H1b) - H0
    xt = x_s[i]
    Xg = jnp.concatenate([H0b, H1b, H2.astype(BF16), xt], axis=1)
    value = jax.nn.sigmoid(
        jnp.dot(Xg, wg_s[...], preferred_element_type=F32) + bg_s[...])
    r = value[:, :BB * RU]
    u = value[:, BB * RU:]

    rH = r * H0
    rHb = rH.astype(BF16)
    R1 = spmm(rHb)
    R1b = R1.astype(BF16)
    R2 = 2.0 * spmm(R1b) - rH
    Xc = jnp.concatenate([rHb, R1b, R2.astype(BF16), xt], axis=1)
    c = jnp.tanh(
        jnp.dot(Xc, wc_s[...], preferred_element_type=F32) + bc_s[...])

    nh = u * H0 + (1.0 - u) * c
    for b in range(BB):
        hout_ref[b] = nh[:, b * RU:(b + 1) * RU]
    pj = jnp.dot(nh.astype(BF16), wp_s[...], preferred_element_type=F32)
    out_ref[0] = pj.T + bp_ref[...]


def kernel(inputs, hidden_state, adj, W_gate, b_gate, W_cand, b_cand,
           W_proj, b_proj):
    xin_t = inputs.T
    h3 = pl.pallas_call(
        _to3d_body,
        grid=(B // 8,),
        in_specs=[pl.BlockSpec((8, N * RU), lambda i: (i, 0))],
        out_specs=pl.BlockSpec((8, N, RU), lambda i: (i, 0, 0)),
        out_shape=jax.ShapeDtypeStruct((B, N, RU), F32),
    )(hidden_state[0])

    wg3 = W_gate.reshape(RU + 1, M, 2 * RU)
    whg = wg3[1:].transpose(1, 0, 2).reshape(M * RU, 2 * RU).astype(BF16)
    wig = wg3[0].astype(BF16)
    wc3 = W_cand.reshape(RU + 1, M, RU)
    whc = wc3[1:].transpose(1, 0, 2).reshape(M * RU, RU).astype(BF16)
    wic = wc3[0].astype(BF16)

    const = lambda i: (0, 0)
    out_bn, hout3 = pl.pallas_call(
        _body,
        grid=(GRID,),
        in_specs=[
            pl.BlockSpec((N, N), const),
            pl.BlockSpec((N, B), const),
            pl.BlockSpec((BB, N, RU), lambda i: (i, 0, 0)),
            pl.BlockSpec((M * RU, 2 * RU), const),
            pl.BlockSpec((M, 2 * RU), const),
            pl.BlockSpec((1, 2 * RU), const),
            pl.BlockSpec((M * RU, RU), const),
            pl.BlockSpec((M, RU), const),
            pl.BlockSpec((1, RU), const),
            pl.BlockSpec((RU, 1), const),
            pl.BlockSpec((1, 1), const),
        ],
        out_specs=[
            pl.BlockSpec((1, BB, N), lambda i: (i, 0, 0)),
            pl.BlockSpec((BB, N, RU), lambda i: (i, 0, 0)),
        ],
        out_shape=[
            jax.ShapeDtypeStruct((GRID, BB, N), F32),
            jax.ShapeDtypeStruct((B, N, RU), F32),
        ],
        scratch_shapes=[
            pltpu.VMEM((N, N), BF16),
            pltpu.VMEM((GRID, N, M * BB), BF16),
            pltpu.VMEM((M * BB * (RU + 1), 2 * BB * RU), BF16),
            pltpu.VMEM((M * BB * (RU + 1), BB * RU), BF16),
            pltpu.VMEM((BB * RU, BB), BF16),
            pltpu.VMEM((1, 2 * BB * RU), F32),
            pltpu.VMEM((1, BB * RU), F32),
        ],
    )(adj, xin_t, h3, whg, wig, b_gate.reshape(1, 2 * RU),
      whc, wic, b_cand.reshape(1, RU), W_proj.astype(BF16),
      b_proj.reshape(1, 1))

    hout2 = pl.pallas_call(
        _to2d_body,
        grid=(B // 8,),
        in_specs=[pl.BlockSpec((8, N, RU), lambda i: (i, 0, 0))],
        out_specs=pl.BlockSpec((8, N * RU), lambda i: (i, 0)),
        out_shape=jax.ShapeDtypeStruct((B, N * RU), F32),
    )(hout3)
    return out_bn.reshape(B, N), hout2[None]
